# Initial kernel scaffold; baseline (speedup 1.0000x reference)
import functools

import jax
import jax.numpy as jnp
from jax import lax
from jax.experimental import pallas as pl
from jax.experimental.pallas import tpu as pltpu
from jax.experimental.pallas import tpu_sc as plsc

H = 128
EPS = 1e-12
NC, NS = 2, 16
NW = NC * NS
IDXW = 128


@functools.partial(jax.jit, static_argnames=("bs",))
def _sc_gather(table, idx2d, bs):
    b_per_w = bs // NW
    k = b_per_w // IDXW
    mesh = plsc.VectorSubcoreMesh(core_axis_name="c", subcore_axis_name="s")

    @functools.partial(
        pl.kernel,
        mesh=mesh,
        out_type=jax.ShapeDtypeStruct((bs, H), jnp.float32),
        scratch_types=[
            pltpu.VMEM((k, IDXW), jnp.int32),
            pltpu.VMEM((IDXW, H), jnp.float32),
            pltpu.VMEM((IDXW, H), jnp.float32),
            pltpu.SemaphoreType.DMA,
            pltpu.SemaphoreType.DMA,
        ],
    )
    def gk(table_hbm, idx_hbm, out_hbm, idx_v, rows_a, rows_b, sem_a, sem_b):
        wid = lax.axis_index("s") * NC + lax.axis_index("c")
        base = wid * b_per_w
        pltpu.sync_copy(idx_hbm.at[pl.ds(wid * k, k)], idx_v)

        def start(j, rows, sem):
            pltpu.async_copy(table_hbm.at[idx_v.at[j]], rows, sem)

        def drain(j, rows, sem):
            pltpu.make_async_copy(table_hbm.at[idx_v.at[j]], rows, sem).wait()
            pltpu.sync_copy(rows, out_hbm.at[pl.ds(base + j * IDXW, IDXW)])

        start(0, rows_a, sem_a)

        @pl.loop(0, k - 2, step=2)
        def _(j):
            start(j + 1, rows_b, sem_b)
            drain(j, rows_a, sem_a)
            start(j + 2, rows_a, sem_a)
            drain(j + 1, rows_b, sem_b)

        start(k - 1, rows_b, sem_b)
        drain(k - 2, rows_a, sem_a)
        drain(k - 1, rows_b, sem_b)

    return gk(table, idx2d)


def _ln_body(g_ref, pos_ref, type_ref, w_ref, b_ref, o_ref):
    x = g_ref[...] + pos_ref[...][None] + type_ref[...][None]
    mean = jnp.mean(x, axis=-1, keepdims=True)
    xc = x - mean
    var = jnp.mean(xc * xc, axis=-1, keepdims=True)
    o_ref[...] = xc * lax.rsqrt(var + EPS) * w_ref[...] + b_ref[...]


@functools.partial(jax.jit, static_argnames=("rb",))
def _tc_ln(gathered, pos, type_row, w, b, rb):
    bsz, s, _ = gathered.shape
    grid = (bsz // rb,)
    return pl.pallas_call(
        _ln_body,
        grid=grid,
        in_specs=[
            pl.BlockSpec((rb, s, H), lambda i: (i, 0, 0)),
            pl.BlockSpec((s, H), lambda i: (0, 0)),
            pl.BlockSpec((1, H), lambda i: (0, 0)),
            pl.BlockSpec((1, H), lambda i: (0, 0)),
            pl.BlockSpec((1, H), lambda i: (0, 0)),
        ],
        out_specs=pl.BlockSpec((rb, s, H), lambda i: (i, 0, 0)),
        out_shape=jax.ShapeDtypeStruct((bsz, s, H), jnp.float32),
    )(gathered, pos, type_row, w, b)


def kernel(input_ids, word_embeddings, position_embeddings,
           token_type_embeddings, ln_weight, ln_bias):
    bsz, s = input_ids.shape
    bs = bsz * s
    idx2d = input_ids.astype(jnp.int32).reshape(bs // IDXW, IDXW)
    gathered = _sc_gather(word_embeddings, idx2d, bs)
    return _tc_ln(
        gathered.reshape(bsz, s, H),
        position_embeddings[:s],
        token_type_embeddings[0:1],
        ln_weight.reshape(1, H),
        ln_bias.reshape(1, H),
        16,
    )


# speedup vs baseline: 4.7747x; 4.7747x over previous
import functools

import jax
import jax.numpy as jnp
from jax import lax
from jax.experimental import pallas as pl
from jax.experimental.pallas import tpu as pltpu
from jax.experimental.pallas import tpu_sc as plsc

H = 128
EPS = 1e-12
NC, NS = 2, 16
NW = NC * NS
IDXW = 128


@functools.partial(jax.jit, static_argnames=("bs",))
def _sc_gather(table, idx3d, bs):
    b_per_w = bs // NW
    k = b_per_w // IDXW
    mesh = plsc.VectorSubcoreMesh(core_axis_name="c", subcore_axis_name="s")

    @functools.partial(
        pl.kernel,
        mesh=mesh,
        out_type=jax.ShapeDtypeStruct((bs, H), jnp.float32),
        scratch_types=[
            pltpu.VMEM((k, IDXW), jnp.int32),
            pltpu.VMEM((IDXW, H), jnp.float32),
            pltpu.VMEM((IDXW, H), jnp.float32),
            pltpu.SemaphoreType.DMA,
            pltpu.SemaphoreType.DMA,
        ],
    )
    def gk(table_hbm, idx_hbm, out_hbm, idx_v, rows_a, rows_b, sem_a, sem_b):
        wid = lax.axis_index("s") * NC + lax.axis_index("c")
        base = wid * b_per_w
        pltpu.sync_copy(idx_hbm.at[wid], idx_v)

        def start(j, rows, sem):
            pltpu.async_copy(table_hbm.at[idx_v.at[j]], rows, sem)

        def drain(j, rows, sem):
            pltpu.make_async_copy(table_hbm.at[idx_v.at[j]], rows, sem).wait()
            pltpu.sync_copy(rows, out_hbm.at[pl.ds(base + j * IDXW, IDXW)])

        start(0, rows_a, sem_a)

        @pl.loop(0, k - 2, step=2)
        def _(j):
            start(j + 1, rows_b, sem_b)
            drain(j, rows_a, sem_a)
            start(j + 2, rows_a, sem_a)
            drain(j + 1, rows_b, sem_b)

        start(k - 1, rows_b, sem_b)
        drain(k - 2, rows_a, sem_a)
        drain(k - 1, rows_b, sem_b)

    return gk(table, idx3d)


def _ln_body(g_ref, pos_ref, type_ref, w_ref, b_ref, o_ref):
    x = g_ref[...] + pos_ref[...][None] + type_ref[...][None]
    mean = jnp.mean(x, axis=-1, keepdims=True)
    xc = x - mean
    var = jnp.mean(xc * xc, axis=-1, keepdims=True)
    o_ref[...] = xc * lax.rsqrt(var + EPS) * w_ref[...] + b_ref[...]


@functools.partial(jax.jit, static_argnames=("rb",))
def _tc_ln(gathered, pos, type_row, w, b, rb):
    bsz, s, _ = gathered.shape
    grid = (bsz // rb,)
    return pl.pallas_call(
        _ln_body,
        grid=grid,
        in_specs=[
            pl.BlockSpec((rb, s, H), lambda i: (i, 0, 0)),
            pl.BlockSpec((s, H), lambda i: (0, 0)),
            pl.BlockSpec((1, H), lambda i: (0, 0)),
            pl.BlockSpec((1, H), lambda i: (0, 0)),
            pl.BlockSpec((1, H), lambda i: (0, 0)),
        ],
        out_specs=pl.BlockSpec((rb, s, H), lambda i: (i, 0, 0)),
        out_shape=jax.ShapeDtypeStruct((bsz, s, H), jnp.float32),
    )(gathered, pos, type_row, w, b)


def kernel(input_ids, word_embeddings, position_embeddings,
           token_type_embeddings, ln_weight, ln_bias):
    bsz, s = input_ids.shape
    bs = bsz * s
    idx3d = input_ids.astype(jnp.int32).reshape(NW, bs // (NW * IDXW), IDXW)
    gathered = _sc_gather(word_embeddings, idx3d, bs)
    return _tc_ln(
        gathered.reshape(bsz, s, H),
        position_embeddings[:s],
        token_type_embeddings[0:1],
        ln_weight.reshape(1, H),
        ln_bias.reshape(1, H),
        16,
    )


# speedup vs baseline: 5.1380x; 1.0761x over previous
import functools

import jax
import jax.numpy as jnp
from jax import lax
from jax.experimental import pallas as pl
from jax.experimental.pallas import tpu as pltpu
from jax.experimental.pallas import tpu_sc as plsc

H = 128
EPS = 1e-12
NC, NS = 2, 16
NW = NC * NS
IDXW = 128


@functools.partial(jax.jit, static_argnames=("bs",))
def _sc_gather(table, idx3d, bs):
    b_per_w = bs // NW
    k = b_per_w // IDXW
    mesh = plsc.VectorSubcoreMesh(core_axis_name="c", subcore_axis_name="s")

    @functools.partial(
        pl.kernel,
        mesh=mesh,
        out_type=jax.ShapeDtypeStruct((bs, H), jnp.float32),
        scratch_types=[
            pltpu.VMEM((k, IDXW), jnp.int32),
            pltpu.VMEM((IDXW, H), jnp.float32),
            pltpu.VMEM((IDXW, H), jnp.float32),
            pltpu.SemaphoreType.DMA,
            pltpu.SemaphoreType.DMA,
        ],
    )
    def gk(table_hbm, idx_hbm, out_hbm, idx_v, rows_a, rows_b, sem_a, sem_b):
        wid = lax.axis_index("s") * NC + lax.axis_index("c")
        base = wid * b_per_w
        pltpu.sync_copy(idx_hbm.at[wid], idx_v)

        def start(j, rows, sem):
            pltpu.async_copy(table_hbm.at[idx_v.at[j]], rows, sem)

        def drain(j, rows, sem):
            pltpu.make_async_copy(table_hbm.at[idx_v.at[j]], rows, sem).wait()
            pltpu.sync_copy(rows, out_hbm.at[pl.ds(base + j * IDXW, IDXW)])

        start(0, rows_a, sem_a)

        @pl.loop(0, k - 2, step=2)
        def _(j):
            start(j + 1, rows_b, sem_b)
            drain(j, rows_a, sem_a)
            start(j + 2, rows_a, sem_a)
            drain(j + 1, rows_b, sem_b)

        start(k - 1, rows_b, sem_b)
        drain(k - 2, rows_a, sem_a)
        drain(k - 1, rows_b, sem_b)

    return gk(table, idx3d)


def _ln_body(g_ref, pt_ref, w_ref, b_ref, o_ref):
    x = g_ref[...] + pt_ref[...][None]
    s1 = jnp.sum(x, axis=-1, keepdims=True)
    s2 = jnp.sum(x * x, axis=-1, keepdims=True)
    mean = s1 * (1.0 / H)
    var = s2 * (1.0 / H) - mean * mean
    o_ref[...] = (x - mean) * lax.rsqrt(var + EPS) * w_ref[...] + b_ref[...]


@functools.partial(jax.jit, static_argnames=("rb",))
def _tc_ln(gathered, pt, w, b, rb):
    bsz, s, _ = gathered.shape
    grid = (bsz // rb,)
    return pl.pallas_call(
        _ln_body,
        grid=grid,
        in_specs=[
            pl.BlockSpec((rb, s, H), lambda i: (i, 0, 0)),
            pl.BlockSpec((s, H), lambda i: (0, 0)),
            pl.BlockSpec((1, H), lambda i: (0, 0)),
            pl.BlockSpec((1, H), lambda i: (0, 0)),
        ],
        out_specs=pl.BlockSpec((rb, s, H), lambda i: (i, 0, 0)),
        out_shape=jax.ShapeDtypeStruct((bsz, s, H), jnp.float32),
        compiler_params=pltpu.CompilerParams(
            dimension_semantics=("arbitrary",)),
    )(gathered, pt, w, b)


def kernel(input_ids, word_embeddings, position_embeddings,
           token_type_embeddings, ln_weight, ln_bias):
    bsz, s = input_ids.shape
    bs = bsz * s
    idx3d = input_ids.astype(jnp.int32).reshape(NW, bs // (NW * IDXW), IDXW)
    gathered = _sc_gather(word_embeddings, idx3d, bs)
    pt = position_embeddings[:s] + token_type_embeddings[0]
    return _tc_ln(
        gathered.reshape(bsz, s, H),
        pt,
        ln_weight.reshape(1, H),
        ln_bias.reshape(1, H),
        32,
    )


# speedup vs baseline: 5.5172x; 1.0738x over previous
import functools

import jax
import jax.numpy as jnp
from jax import lax
from jax.experimental import pallas as pl
from jax.experimental.pallas import tpu as pltpu
from jax.experimental.pallas import tpu_sc as plsc

H = 128
EPS = 1e-12
NC, NS = 2, 16
NW = NC * NS
IDXW = 128
RING = 5
PIECES = 2
RB = 32


@functools.partial(jax.jit, static_argnames=("bs",))
def _sc_gather(table, idx3d, bs):
    b_per_w = bs // NW
    k = b_per_w // IDXW
    assert k % RING == 0 and k >= 2 * RING
    mesh = plsc.VectorSubcoreMesh(core_axis_name="c", subcore_axis_name="s")

    @functools.partial(
        pl.kernel,
        mesh=mesh,
        out_type=jax.ShapeDtypeStruct((bs, H), jnp.float32),
        scratch_types=(
            [pltpu.VMEM((k, IDXW), jnp.int32)]
            + [pltpu.VMEM((IDXW, H), jnp.float32) for _ in range(RING)]
            + [pltpu.SemaphoreType.DMA for _ in range(RING)]
        ),
    )
    def gk(table_hbm, idx_hbm, out_hbm, idx_v, *bufs_sems):
        rows = bufs_sems[:RING]
        sems = bufs_sems[RING:]
        wid = lax.axis_index("s") * NC + lax.axis_index("c")
        base = wid * b_per_w
        pltpu.sync_copy(idx_hbm.at[wid], idx_v)

        def start(j, b):
            pltpu.async_copy(table_hbm.at[idx_v.at[j]], rows[b], sems[b])

        def drain(j, b):
            pltpu.make_async_copy(
                table_hbm.at[idx_v.at[j]], rows[b], sems[b]).wait()
            pltpu.sync_copy(
                rows[b], out_hbm.at[pl.ds(base + j * IDXW, IDXW)])

        for b in range(RING):
            start(b, b)

        @pl.loop(0, k - RING, step=RING)
        def _(j):
            for b in range(RING):
                drain(j + b, b)
                start(j + b + RING, b)

        for b in range(RING):
            drain(k - RING + b, b)

    return gk(table, idx3d)


def _ln_body(*refs):
    g_ref, pt_ref, w_ref, b_ref = refs[:4]
    o_ref = refs[-1]
    x = g_ref[...] + pt_ref[...][None]
    s1 = jnp.sum(x, axis=-1, keepdims=True)
    s2 = jnp.sum(x * x, axis=-1, keepdims=True)
    mean = s1 * (1.0 / H)
    var = s2 * (1.0 / H) - mean * mean
    o_ref[...] = (x - mean) * lax.rsqrt(var + EPS) * w_ref[...] + b_ref[...]


@functools.partial(jax.jit, static_argnames=("piece", "full_bsz"))
def _tc_ln_piece(gathered, pt, w, b, prev, piece, full_bsz):
    pbsz, s, _ = gathered.shape
    grid = (pbsz // RB,)
    row0 = piece * (pbsz // RB)
    in_specs = [
        pl.BlockSpec((RB, s, H), lambda i: (i, 0, 0)),
        pl.BlockSpec((s, H), lambda i: (0, 0)),
        pl.BlockSpec((1, H), lambda i: (0, 0)),
        pl.BlockSpec((1, H), lambda i: (0, 0)),
    ]
    args = [gathered, pt, w, b]
    aliases = {}
    if prev is not None:
        in_specs.append(pl.BlockSpec(memory_space=pl.ANY))
        args.append(prev)
        aliases = {4: 0}
    return pl.pallas_call(
        _ln_body,
        grid=grid,
        in_specs=in_specs,
        out_specs=pl.BlockSpec((RB, s, H), lambda i: (row0 + i, 0, 0)),
        out_shape=jax.ShapeDtypeStruct((full_bsz, s, H), jnp.float32),
        input_output_aliases=aliases,
        compiler_params=pltpu.CompilerParams(
            dimension_semantics=("arbitrary",)),
    )(*args)


def kernel(input_ids, word_embeddings, position_embeddings,
           token_type_embeddings, ln_weight, ln_bias):
    bsz, s = input_ids.shape
    bs = bsz * s
    ids = input_ids.astype(jnp.int32).reshape(-1)
    pt = position_embeddings[:s] + token_type_embeddings[0]
    w = ln_weight.reshape(1, H)
    b = ln_bias.reshape(1, H)

    pbs = bs // PIECES
    pbsz = bsz // PIECES
    k = pbs // (NW * IDXW)
    out = None
    for p in range(PIECES):
        idx3d = ids[p * pbs:(p + 1) * pbs].reshape(NW, k, IDXW)
        g = _sc_gather(word_embeddings, idx3d, pbs)
        out = _tc_ln_piece(g.reshape(pbsz, s, H), pt, w, b, out,
                           piece=p, full_bsz=bsz)
    return out
